# Initial kernel scaffold; baseline (speedup 1.0000x reference)
import functools

import jax
import jax.numpy as jnp
from jax import lax
from jax.experimental import pallas as pl
from jax.experimental.pallas import tpu as pltpu
from jax.experimental.pallas import tpu_sc as plsc

NC = 2
NS = 16
NW = NC * NS
L = 16


def _make_sc_segment(n, e, d):
    ew = e // NW
    c = 80
    nch = ew // c
    rps = n // NS
    nfull = rps // c
    rem = rps - nfull * c
    nj = d // L

    mesh = plsc.VectorSubcoreMesh(
        core_axis_name="c", subcore_axis_name="s",
        num_cores=NC, num_subcores=NS)

    @functools.partial(
        pl.kernel,
        out_type=jax.ShapeDtypeStruct((NC, n, d), jnp.float32),
        mesh=mesh,
        scratch_types=[
            pltpu.VMEM((c,), jnp.int32),
            pltpu.VMEM((c,), jnp.int32),
            pltpu.VMEM((c,), jnp.float32),
            pltpu.VMEM((c, d), jnp.float32),
            pltpu.VMEM_SHARED((n, d), jnp.float32),
            pltpu.SemaphoreType.DMA,
        ],
    )
    def sc_segment(x_hbm, src_hbm, dst_hbm, w_hbm, out_hbm,
                   src_v, dst_v, w_v, rows_v, acc, sem):
        cid = lax.axis_index("c")
        sid = lax.axis_index("s")
        wid = sid * NC + cid

        zero16 = jnp.zeros((L,), jnp.float32)

        def zrow(i, carry):
            for j in range(nj):
                rows_v[i, pl.ds(j * L, L)] = zero16
            return carry

        lax.fori_loop(0, c, zrow, 0)

        rbase = sid * rps
        for i in range(nfull):
            pltpu.sync_copy(rows_v, acc.at[pl.ds(rbase + i * c, c)])
        if rem:
            pltpu.sync_copy(rows_v.at[pl.ds(0, rem)],
                            acc.at[pl.ds(rbase + nfull * c, rem)])
        plsc.subcore_barrier()

        ebase = wid * ew

        def chunk(i, carry):
            off = ebase + i * c
            pltpu.sync_copy(src_hbm.at[pl.ds(off, c)], src_v)
            pltpu.sync_copy(dst_hbm.at[pl.ds(off, c)], dst_v)
            pltpu.sync_copy(w_hbm.at[pl.ds(off, c)], w_v)
            pltpu.async_copy(x_hbm.at[src_v], rows_v, sem).wait()

            def wmul(ei, carry2):
                wb = plsc.load_gather(w_v, [jnp.full((L,), ei, jnp.int32)])
                for j in range(nj):
                    sl = pl.ds(j * L, L)
                    rows_v[ei, sl] = rows_v[ei, sl] * wb
                return carry2

            lax.fori_loop(0, c, wmul, 0)

            pltpu.sync_copy(rows_v, acc.at[dst_v], add=True)
            return carry

        lax.fori_loop(0, nch, chunk, 0)
        plsc.subcore_barrier()

        for i in range(nfull):
            sl = pl.ds(rbase + i * c, c)
            pltpu.sync_copy(acc.at[sl], out_hbm.at[cid, sl])
        if rem:
            sl = pl.ds(rbase + nfull * c, rem)
            pltpu.sync_copy(acc.at[sl], out_hbm.at[cid, sl])

    return sc_segment


def _tc_finish_body(x_ref, f_ref, w_ref, b_ref, o_ref):
    h = x_ref[...] + f_ref[0] + f_ref[1]
    y = lax.dot_general(h, w_ref[...], (((1,), (1,)), ((), ())),
                        preferred_element_type=jnp.float32)
    o_ref[...] = jnp.tanh(y + b_ref[...])


def _make_tc_finish(n, d, br):
    return pl.pallas_call(
        _tc_finish_body,
        grid=(n // br,),
        in_specs=[
            pl.BlockSpec((br, d), lambda i: (i, 0)),
            pl.BlockSpec((NC, br, d), lambda i: (0, i, 0)),
            pl.BlockSpec((d, d), lambda i: (0, 0)),
            pl.BlockSpec((1, d), lambda i: (0, 0)),
        ],
        out_specs=pl.BlockSpec((br, d), lambda i: (i, 0)),
        out_shape=jax.ShapeDtypeStruct((n, d), jnp.float32),
    )


@jax.jit
def kernel(x, edge_index, edge_weight, W, b):
    n, d = x.shape
    e = edge_index.shape[1]
    src = edge_index[0]
    dst = edge_index[1]
    ft_partial = _make_sc_segment(n, e, d)(x, src, dst, edge_weight)
    return _make_tc_finish(n, d, 400)(x, ft_partial, W, b.reshape(1, d))


# speedup vs baseline: 4.1155x; 4.1155x over previous
import functools

import jax
import jax.numpy as jnp
from jax import lax
from jax.experimental import pallas as pl
from jax.experimental.pallas import tpu as pltpu
from jax.experimental.pallas import tpu_sc as plsc

NC = 2
NS = 16
NW = NC * NS
L = 16

_BCAST_DNUMS = lax.GatherDimensionNumbers(
    offset_dims=(), collapsed_slice_dims=(0,), start_index_map=(0,))


def _lane_bcast(vec, k):
    idx = jnp.full((L, 1), k, jnp.int32)
    return lax.gather(vec, idx, _BCAST_DNUMS, (1,),
                      mode=lax.GatherScatterMode.PROMISE_IN_BOUNDS)


def _make_sc_segment(n, e, d):
    ew = e // NW
    c = 80
    nch = ew // c
    stripe = (n // NS) & ~7
    tail = n - NS * stripe
    nj = d // L

    mesh = plsc.VectorSubcoreMesh(
        core_axis_name="c", subcore_axis_name="s",
        num_cores=NC, num_subcores=NS)

    @functools.partial(
        pl.kernel,
        out_type=jax.ShapeDtypeStruct((NC, n, d), jnp.float32),
        mesh=mesh,
        scratch_types=[
            pltpu.VMEM((c,), jnp.int32),
            pltpu.VMEM((c,), jnp.int32),
            pltpu.VMEM((c,), jnp.float32),
            pltpu.VMEM((c, d), jnp.float32),
            pltpu.VMEM_SHARED((n, d), jnp.float32),
            pltpu.SemaphoreType.DMA,
        ],
    )
    def sc_segment(x_hbm, src_hbm, dst_hbm, w_hbm, out_hbm,
                   src_v, dst_v, w_v, rows_v, acc, sem):
        cid = lax.axis_index("c")
        sid = lax.axis_index("s")
        wid = sid * NC + cid

        zero16 = jnp.zeros((L,), jnp.float32)

        def zrow(i, carry):
            for j in range(nj):
                rows_v[i, pl.ds(j * L, L)] = zero16
            return carry

        lax.fori_loop(0, c, zrow, 0)

        def copy_rows(src_fn, dst_fn, base, count):
            done = 0
            while count - done >= c:
                src_fn_c, dst_fn_c = src_fn(base + done, c), dst_fn(base + done, c)
                pltpu.sync_copy(src_fn_c, dst_fn_c)
                done += c
            if count - done:
                r = count - done
                pltpu.sync_copy(src_fn(base + done, r), dst_fn(base + done, r))

        rbase = pl.multiple_of(sid * stripe, 8)

        def zsrc(off, cnt):
            return rows_v.at[pl.ds(0, cnt)]

        copy_rows(zsrc, lambda off, cnt: acc.at[pl.ds(off, cnt)], rbase, stripe)

        @pl.when(sid == NS - 1)
        def _():
            copy_rows(zsrc, lambda off, cnt: acc.at[pl.ds(off, cnt)],
                      NS * stripe, tail)

        plsc.subcore_barrier()

        ebase = wid * ew

        def chunk(i, carry):
            off = ebase + i * c
            pltpu.sync_copy(src_hbm.at[pl.ds(off, c)], src_v)
            pltpu.sync_copy(dst_hbm.at[pl.ds(off, c)], dst_v)
            pltpu.sync_copy(w_hbm.at[pl.ds(off, c)], w_v)
            pltpu.async_copy(x_hbm.at[src_v], rows_v, sem).wait()

            def wmul(g, carry2):
                wg = w_v[pl.ds(g * L, L)]
                for k in range(L):
                    wb = _lane_bcast(wg, k)
                    ei = g * L + k
                    for j in range(nj):
                        sl = pl.ds(j * L, L)
                        rows_v[ei, sl] = rows_v[ei, sl] * wb
                return carry2

            lax.fori_loop(0, c // L, wmul, 0)

            pltpu.sync_copy(rows_v, acc.at[dst_v], add=True)
            return carry

        lax.fori_loop(0, nch, chunk, 0)
        plsc.subcore_barrier()

        def asrc(off, cnt):
            return acc.at[pl.ds(off, cnt)]

        def odst(off, cnt):
            return out_hbm.at[cid, pl.ds(off, cnt)]

        copy_rows(asrc, odst, rbase, stripe)

        @pl.when(sid == NS - 1)
        def _():
            copy_rows(asrc, odst, NS * stripe, tail)

    return sc_segment


def _tc_finish_body(x_ref, f_ref, w_ref, b_ref, o_ref):
    h = x_ref[...] + f_ref[0] + f_ref[1]
    y = lax.dot_general(h, w_ref[...], (((1,), (1,)), ((), ())),
                        preferred_element_type=jnp.float32)
    o_ref[...] = jnp.tanh(y + b_ref[...])


def _make_tc_finish(n, d, br):
    return pl.pallas_call(
        _tc_finish_body,
        grid=(n // br,),
        in_specs=[
            pl.BlockSpec((br, d), lambda i: (i, 0)),
            pl.BlockSpec((NC, br, d), lambda i: (0, i, 0)),
            pl.BlockSpec((d, d), lambda i: (0, 0)),
            pl.BlockSpec((1, d), lambda i: (0, 0)),
        ],
        out_specs=pl.BlockSpec((br, d), lambda i: (i, 0)),
        out_shape=jax.ShapeDtypeStruct((n, d), jnp.float32),
    )


@jax.jit
def kernel(x, edge_index, edge_weight, W, b):
    n, d = x.shape
    e = edge_index.shape[1]
    src = edge_index[0]
    dst = edge_index[1]
    ft_partial = _make_sc_segment(n, e, d)(x, src, dst, edge_weight)
    return _make_tc_finish(n, d, 400)(x, ft_partial, W, b.reshape(1, d))
